# baseline (speedup 1.0000x reference)
import functools

import jax
import jax.numpy as jnp
from jax import lax
from jax.experimental import pallas as pl
from jax.experimental.pallas import tpu as pltpu
from jax.experimental.pallas import tpu_sc as plsc

_E = 0.1
_NW = 32
_LANE = 16


def _dense_body(x_ref, m_ref, slse_ref, colsum_ref, *, nb, br, c):
    i = pl.program_id(0)

    @pl.when(i == 0)
    def _init():
        slse_ref[...] = jnp.zeros_like(slse_ref)
        colsum_ref[...] = jnp.zeros_like(colsum_ref)

    xb = x_ref[...]
    m = jnp.max(xb, axis=1, keepdims=True)
    em = jnp.exp(xb - m)
    s = jnp.dot(em, jnp.ones((c, 1), jnp.float32),
                preferred_element_type=jnp.float32)
    lse = m + jnp.log(s)
    m_ref[...] = m
    slse_ref[...] += jnp.sum(lse, axis=0, keepdims=True)
    colsum_ref[...] += jnp.dot(jnp.ones((1, br), jnp.float32), xb,
                               preferred_element_type=jnp.float32)


def _dense(x):
    b, c = x.shape
    br = 512
    nb = b // br
    return pl.pallas_call(
        functools.partial(_dense_body, nb=nb, br=br, c=c),
        grid=(nb,),
        in_specs=[pl.BlockSpec((br, c), lambda i: (i, 0))],
        out_specs=[
            pl.BlockSpec((br, 1), lambda i: (i, 0)),
            pl.BlockSpec((1, 1), lambda i: (0, 0)),
            pl.BlockSpec((1, c), lambda i: (0, 0)),
        ],
        out_shape=[
            jax.ShapeDtypeStruct((b, 1), jnp.float32),
            jax.ShapeDtypeStruct((1, 1), jnp.float32),
            jax.ShapeDtypeStruct((1, c), jnp.float32),
        ],
    )(x)


def _sc_stage(xflat, target, mflat, b, c):
    rows = b // _NW
    nch = rows // _LANE
    cpad = 1024
    mesh = plsc.VectorSubcoreMesh(core_axis_name="cc", subcore_axis_name="sc")

    @functools.partial(
        pl.kernel, mesh=mesh,
        compiler_params=pltpu.CompilerParams(needs_layout_passes=False),
        out_type=[
            jax.ShapeDtypeStruct((_NW, cpad), jnp.float32),
            jax.ShapeDtypeStruct((_NW, cpad), jnp.float32),
            jax.ShapeDtypeStruct((_NW, _LANE), jnp.float32),
        ],
        scratch_types=[
            pltpu.VMEM((rows,), jnp.int32),
            pltpu.VMEM((rows,), jnp.float32),
            pltpu.VMEM((4, rows // 4), jnp.int32),
            pltpu.VMEM((4, rows // 4), jnp.float32),
            pltpu.VMEM((cpad,), jnp.float32),
            pltpu.VMEM((cpad,), jnp.float32),
            pltpu.VMEM((_LANE,), jnp.float32),
            pltpu.SemaphoreType.DMA,
        ],
    )
    def sc_kernel(x_hbm, t_hbm, m_hbm, counts_out, corr_out, sxt_out,
                  tgt_v, m_v, fidx_v, xt_v, cnt_v, cor_v, acc_v, sem):
        wid = lax.axis_index("sc") * 2 + lax.axis_index("cc")
        base = wid * rows

        pltpu.sync_copy(t_hbm.at[pl.ds(base, rows)], tgt_v)
        pltpu.sync_copy(m_hbm.at[pl.ds(base, rows)], m_v)

        per_row = rows // 4
        zero16 = jnp.zeros((_LANE,), jnp.float32)
        for k in range(cpad // _LANE):
            cnt_v[pl.ds(k * _LANE, _LANE)] = zero16
            cor_v[pl.ds(k * _LANE, _LANE)] = zero16

        lane = lax.iota(jnp.int32, _LANE)
        for k in range(nch):
            t16 = tgt_v[pl.ds(k * _LANE, _LANE)]
            row = (base + k * _LANE) + lane
            fidx_v[k * _LANE // per_row,
                   pl.ds((k * _LANE) % per_row, _LANE)] = row * c + t16

        for j in range(4):
            pltpu.async_copy(x_hbm.at[fidx_v.at[j]], xt_v.at[j], sem).wait()

        ones16 = jnp.ones((_LANE,), jnp.float32)
        sxt = zero16
        for k in range(nch):
            t16 = tgt_v[pl.ds(k * _LANE, _LANE)]
            xt16 = xt_v[k * _LANE // per_row, pl.ds((k * _LANE) % per_row, _LANE)]
            m16 = m_v[pl.ds(k * _LANE, _LANE)]
            corr16 = jnp.where(xt16 >= m16, 1.0, 0.0).astype(jnp.float32)
            sxt = sxt + xt16
            plsc.addupdate_scatter(cnt_v, [t16], ones16)
            plsc.addupdate_scatter(cor_v, [t16], corr16)
        acc_v[...] = sxt

        pltpu.sync_copy(cnt_v, counts_out.at[wid])
        pltpu.sync_copy(cor_v, corr_out.at[wid])
        pltpu.sync_copy(acc_v, sxt_out.at[wid])

    return sc_kernel(xflat, target, mflat)


def _comb_body(cntp_ref, corp_ref, col_ref, slse_ref, sxtp_ref, out_ref,
               *, b, c, cpad):
    counts = jnp.sum(cntp_ref[...], axis=0, keepdims=True)
    corr = jnp.sum(corp_ref[...], axis=0, keepdims=True)
    sxt = jnp.sum(sxtp_ref[...], keepdims=True).reshape(1, 1)
    acc = corr / jnp.maximum(counts, 1.0)
    cw = 1.0 / (acc + 1e-5)
    mask = lax.broadcasted_iota(jnp.int32, (1, cpad), 1) < c
    cw = jnp.where(mask, cw, 0.0)
    cw_sum = jnp.sum(cw, axis=1, keepdims=True)
    dot = jnp.sum(cw[:, :c] * col_ref[...], axis=1, keepdims=True)
    smooth = _E / c
    out_ref[...] = ((0.9 + smooth) * slse_ref[...]
                    - 0.9 * sxt
                    - smooth * dot / cw_sum) / b


def _combine(counts_p, corr_p, colsum, slse, sxt_p, b, c):
    cpad = counts_p.shape[1]
    return pl.pallas_call(
        functools.partial(_comb_body, b=b, c=c, cpad=cpad),
        out_shape=jax.ShapeDtypeStruct((1, 1), jnp.float32),
    )(counts_p, corr_p, colsum, slse, sxt_p)


def kernel(x, target):
    b, c = x.shape
    m2, slse, colsum = _dense(x)
    counts_p, corr_p, sxt_p = _sc_stage(x.reshape(b * c), target,
                                        m2.reshape(b), b, c)
    out = _combine(counts_p, corr_p, colsum, slse, sxt_p, b, c)
    return out[0, 0]


# speedup vs baseline: 1.5173x; 1.5173x over previous
import functools

import jax
import jax.numpy as jnp
from jax import lax
from jax.experimental import pallas as pl
from jax.experimental.pallas import tpu as pltpu
from jax.experimental.pallas import tpu_sc as plsc

_E = 0.1
_NW = 32
_LANE = 16


def _dense_body(x_ref, t_ref, corr_ref, slse_ref, sxt_ref, colsum_ref,
                *, nb, br, c):
    i = pl.program_id(0)

    @pl.when(i == 0)
    def _init():
        slse_ref[...] = jnp.zeros_like(slse_ref)
        sxt_ref[...] = jnp.zeros_like(sxt_ref)
        colsum_ref[...] = jnp.zeros_like(colsum_ref)

    xb = x_ref[...]
    tb = t_ref[...]
    iota = lax.broadcasted_iota(jnp.int32, (br, c), 1)

    m = jnp.max(xb, axis=1, keepdims=True)
    em = jnp.exp(xb - m)
    sel = jnp.where(iota == tb, xb, 0.0)
    ones_c1 = jnp.ones((c, 1), jnp.float32)
    s = jnp.dot(em, ones_c1, preferred_element_type=jnp.float32)
    xt = jnp.sum(sel, axis=1, keepdims=True)
    lse = m + jnp.log(s)
    corr_ref[...] = (xt == m).astype(jnp.float32)
    slse_ref[...] += jnp.sum(lse, axis=0, keepdims=True)
    sxt_ref[...] += jnp.sum(xt, axis=0, keepdims=True)
    colsum_ref[...] += jnp.dot(jnp.ones((1, br), jnp.float32), xb,
                               preferred_element_type=jnp.float32)


def _dense(x, t2):
    b, c = x.shape
    br = 512
    nb = b // br
    return pl.pallas_call(
        functools.partial(_dense_body, nb=nb, br=br, c=c),
        grid=(nb,),
        in_specs=[
            pl.BlockSpec((br, c), lambda i: (i, 0)),
            pl.BlockSpec((br, 1), lambda i: (i, 0)),
        ],
        out_specs=[
            pl.BlockSpec((br, 1), lambda i: (i, 0)),
            pl.BlockSpec((1, 1), lambda i: (0, 0)),
            pl.BlockSpec((1, 1), lambda i: (0, 0)),
            pl.BlockSpec((1, c), lambda i: (0, 0)),
        ],
        out_shape=[
            jax.ShapeDtypeStruct((b, 1), jnp.float32),
            jax.ShapeDtypeStruct((1, 1), jnp.float32),
            jax.ShapeDtypeStruct((1, 1), jnp.float32),
            jax.ShapeDtypeStruct((1, c), jnp.float32),
        ],
    )(x, t2)


def _sc_stage(target, correct, b):
    rows = b // _NW
    nch = rows // _LANE
    cpad = 1024
    mesh = plsc.VectorSubcoreMesh(core_axis_name="cc", subcore_axis_name="sc")

    @functools.partial(
        pl.kernel, mesh=mesh,
        compiler_params=pltpu.CompilerParams(needs_layout_passes=False),
        out_type=[
            jax.ShapeDtypeStruct((_NW, cpad), jnp.float32),
            jax.ShapeDtypeStruct((_NW, cpad), jnp.float32),
        ],
        scratch_types=[
            pltpu.VMEM((rows,), jnp.int32),
            pltpu.VMEM((rows,), jnp.float32),
            pltpu.VMEM((cpad,), jnp.float32),
            pltpu.VMEM((cpad,), jnp.float32),
        ],
    )
    def sc_kernel(t_hbm, c_hbm, counts_out, corr_out,
                  tgt_v, cor_in_v, cnt_v, cor_v):
        wid = lax.axis_index("sc") * 2 + lax.axis_index("cc")
        base = wid * rows

        pltpu.sync_copy(t_hbm.at[pl.ds(base, rows)], tgt_v)
        pltpu.sync_copy(c_hbm.at[pl.ds(base, rows)], cor_in_v)

        zero16 = jnp.zeros((_LANE,), jnp.float32)
        for k in range(cpad // _LANE):
            cnt_v[pl.ds(k * _LANE, _LANE)] = zero16
            cor_v[pl.ds(k * _LANE, _LANE)] = zero16

        ones16 = jnp.ones((_LANE,), jnp.float32)
        for k in range(nch):
            t16 = tgt_v[pl.ds(k * _LANE, _LANE)]
            c16 = cor_in_v[pl.ds(k * _LANE, _LANE)]
            plsc.addupdate_scatter(cnt_v, [t16], ones16)
            plsc.addupdate_scatter(cor_v, [t16], c16)

        pltpu.sync_copy(cnt_v, counts_out.at[wid])
        pltpu.sync_copy(cor_v, corr_out.at[wid])

    return sc_kernel(target, correct)


def _comb_body(cntp_ref, corp_ref, col_ref, slse_ref, sxt_ref, out_ref,
               *, b, c, cpad):
    counts = jnp.sum(cntp_ref[...], axis=0, keepdims=True)
    corr = jnp.sum(corp_ref[...], axis=0, keepdims=True)
    acc = corr / jnp.maximum(counts, 1.0)
    cw = 1.0 / (acc + 1e-5)
    mask = lax.broadcasted_iota(jnp.int32, (1, cpad), 1) < c
    cw = jnp.where(mask, cw, 0.0)
    cw_sum = jnp.sum(cw, axis=1, keepdims=True)
    dot = jnp.sum(cw[:, :c] * col_ref[...], axis=1, keepdims=True)
    smooth = _E / c
    out_ref[...] = ((0.9 + smooth) * slse_ref[...]
                    - 0.9 * sxt_ref[...]
                    - smooth * dot / cw_sum) / b


def _combine(counts_p, corr_p, colsum, slse, sxt, b, c):
    cpad = counts_p.shape[1]
    return pl.pallas_call(
        functools.partial(_comb_body, b=b, c=c, cpad=cpad),
        out_shape=jax.ShapeDtypeStruct((1, 1), jnp.float32),
    )(counts_p, corr_p, colsum, slse, sxt)


def kernel(x, target):
    b, c = x.shape
    correct, slse, sxt, colsum = _dense(x, target.reshape(b, 1))
    counts_p, corr_p = _sc_stage(target, correct.reshape(b), b)
    out = _combine(counts_p, corr_p, colsum, slse, sxt, b, c)
    return out[0, 0]


# speedup vs baseline: 1.6995x; 1.1201x over previous
import functools

import jax
import jax.numpy as jnp
from jax import lax
from jax.experimental import pallas as pl
from jax.experimental.pallas import tpu as pltpu
from jax.experimental.pallas import tpu_sc as plsc

_E = 0.1
_NW = 32
_LANE = 16


def _dense_body(x_ref, t_ref, corr_ref, slse_ref, sxt_ref, colsum_ref,
                *, nb, br, c):
    i = pl.program_id(0)

    @pl.when(i == 0)
    def _init():
        slse_ref[...] = jnp.zeros_like(slse_ref)
        sxt_ref[...] = jnp.zeros_like(sxt_ref)
        colsum_ref[...] = jnp.zeros_like(colsum_ref)

    xb = x_ref[...]
    tb = t_ref[...]
    iota = lax.broadcasted_iota(jnp.int32, (br, c), 1)

    m = jnp.max(xb, axis=1, keepdims=True)
    em = jnp.exp(xb - m)
    sel = jnp.where(iota == tb, xb, 0.0)
    ones_c1 = jnp.ones((c, 1), jnp.float32)
    s = jnp.dot(em, ones_c1, preferred_element_type=jnp.float32)
    xt = jnp.sum(sel, axis=1, keepdims=True)
    lse = m + jnp.log(s)
    corr_ref[...] = (xt == m).astype(jnp.float32)
    slse_ref[...] += jnp.sum(lse, axis=0, keepdims=True)
    sxt_ref[...] += jnp.sum(xt, axis=0, keepdims=True)
    colsum_ref[...] += jnp.dot(jnp.ones((1, br), jnp.float32), xb,
                               preferred_element_type=jnp.float32)


def _dense(x, t2):
    b, c = x.shape
    br = 2048
    nb = b // br
    return pl.pallas_call(
        functools.partial(_dense_body, nb=nb, br=br, c=c),
        grid=(nb,),
        in_specs=[
            pl.BlockSpec((br, c), lambda i: (i, 0)),
            pl.BlockSpec((br, 1), lambda i: (i, 0)),
        ],
        out_specs=[
            pl.BlockSpec((br, 1), lambda i: (i, 0)),
            pl.BlockSpec((1, 1), lambda i: (0, 0)),
            pl.BlockSpec((1, 1), lambda i: (0, 0)),
            pl.BlockSpec((1, c), lambda i: (0, 0)),
        ],
        out_shape=[
            jax.ShapeDtypeStruct((b, 1), jnp.float32),
            jax.ShapeDtypeStruct((1, 1), jnp.float32),
            jax.ShapeDtypeStruct((1, 1), jnp.float32),
            jax.ShapeDtypeStruct((1, c), jnp.float32),
        ],
    )(x, t2)


def _sc_stage(target, correct, b):
    rows = b // _NW
    nch = rows // _LANE
    cpad = 1024
    mesh = plsc.VectorSubcoreMesh(core_axis_name="cc", subcore_axis_name="sc")

    @functools.partial(
        pl.kernel, mesh=mesh,
        compiler_params=pltpu.CompilerParams(needs_layout_passes=False),
        out_type=[
            jax.ShapeDtypeStruct((_NW, cpad), jnp.float32),
            jax.ShapeDtypeStruct((_NW, cpad), jnp.float32),
        ],
        scratch_types=[
            pltpu.VMEM((rows,), jnp.int32),
            pltpu.VMEM((rows,), jnp.float32),
            pltpu.VMEM((cpad,), jnp.float32),
            pltpu.VMEM((cpad,), jnp.float32),
        ],
    )
    def sc_kernel(t_hbm, c_hbm, counts_out, corr_out,
                  tgt_v, cor_in_v, cnt_v, cor_v):
        wid = lax.axis_index("sc") * 2 + lax.axis_index("cc")
        base = wid * rows

        pltpu.sync_copy(t_hbm.at[pl.ds(base, rows)], tgt_v)
        pltpu.sync_copy(c_hbm.at[pl.ds(base, rows)], cor_in_v)

        zero16 = jnp.zeros((_LANE,), jnp.float32)
        for k in range(cpad // _LANE):
            cnt_v[pl.ds(k * _LANE, _LANE)] = zero16
            cor_v[pl.ds(k * _LANE, _LANE)] = zero16

        ones16 = jnp.ones((_LANE,), jnp.float32)
        for k in range(nch):
            t16 = tgt_v[pl.ds(k * _LANE, _LANE)]
            c16 = cor_in_v[pl.ds(k * _LANE, _LANE)]
            plsc.addupdate_scatter(cnt_v, [t16], ones16)
            plsc.addupdate_scatter(cor_v, [t16], c16)

        pltpu.sync_copy(cnt_v, counts_out.at[wid])
        pltpu.sync_copy(cor_v, corr_out.at[wid])

    return sc_kernel(target, correct)


def _comb_body(cntp_ref, corp_ref, col_ref, slse_ref, sxt_ref, out_ref,
               *, b, c, cpad):
    counts = jnp.sum(cntp_ref[...], axis=0, keepdims=True)
    corr = jnp.sum(corp_ref[...], axis=0, keepdims=True)
    acc = corr / jnp.maximum(counts, 1.0)
    cw = 1.0 / (acc + 1e-5)
    mask = lax.broadcasted_iota(jnp.int32, (1, cpad), 1) < c
    cw = jnp.where(mask, cw, 0.0)
    cw_sum = jnp.sum(cw, axis=1, keepdims=True)
    dot = jnp.sum(cw[:, :c] * col_ref[...], axis=1, keepdims=True)
    smooth = _E / c
    out_ref[...] = ((0.9 + smooth) * slse_ref[...]
                    - 0.9 * sxt_ref[...]
                    - smooth * dot / cw_sum) / b


def _combine(counts_p, corr_p, colsum, slse, sxt, b, c):
    cpad = counts_p.shape[1]
    return pl.pallas_call(
        functools.partial(_comb_body, b=b, c=c, cpad=cpad),
        out_shape=jax.ShapeDtypeStruct((1, 1), jnp.float32),
    )(counts_p, corr_p, colsum, slse, sxt)


def kernel(x, target):
    b, c = x.shape
    correct, slse, sxt, colsum = _dense(x, target.reshape(b, 1))
    counts_p, corr_p = _sc_stage(target, correct.reshape(b), b)
    out = _combine(counts_p, corr_p, colsum, slse, sxt, b, c)
    return out[0, 0]
